# baseline (speedup 1.0000x reference)
import functools
import math

import jax
import jax.numpy as jnp
import numpy as np
from jax import lax
from jax.experimental import pallas as pl
from jax.experimental.pallas import tpu as pltpu
from jax.experimental.pallas import tpu_sc as plsc

_NUM_BUCKETS = 100
_EMBED_DIM = 128
_MAX_VAL = 1000000.0
_LOG_SCALE = (_NUM_BUCKETS - 1) / math.log(_MAX_VAL)
_I32_MAX = np.int32(2**31 - 1)
_I32_MIN = np.int32(-(2**31))

_BASES = np.round(
    np.exp(np.arange(_NUM_BUCKETS) * (math.log(_MAX_VAL) / (_NUM_BUCKETS - 1)))
).astype(np.int64)
_CANDS = (_BASES[:, None] + np.arange(-16, 16)[None, :]).astype(np.int32)


def _quantize_f32(t_i32):
    clamped = jnp.clip(t_i32.astype(jnp.float32), 1.0, None)
    log_times = jnp.log(clamped) / math.log(_MAX_VAL) * (_NUM_BUCKETS - 1)
    return jnp.clip(log_times.astype(jnp.int32), 0, _NUM_BUCKETS - 1)


def _thresholds():
    cands = lax.optimization_barrier(jnp.asarray(_CANDS))
    q = _quantize_f32(cands)
    ok = q >= jnp.arange(_NUM_BUCKETS, dtype=jnp.int32)[:, None]
    th = jnp.min(jnp.where(ok, cands, _I32_MAX), axis=1).astype(jnp.int32)
    return th.at[0].set(_I32_MIN)


def _project_tables_kernel(tabs_ref, w3_ref, b_ref, out_ref):
    for p in range(3):
        acc = lax.dot_general(
            tabs_ref[p], w3_ref[p], (((1,), (1,)), ((), ())),
            preferred_element_type=jnp.float32)
        if p == 0:
            acc = acc + b_ref[:]
        out_ref[p] = acc


def _projected_tables(abs_tab, rel_tab, sess_tab, W, b):
    d3 = _EMBED_DIM // 3

    def pad_tab(t):
        return jnp.pad(t, ((0, _EMBED_DIM - _NUM_BUCKETS), (0, 48 - t.shape[1])))

    tabs = jnp.stack([pad_tab(abs_tab), pad_tab(rel_tab), pad_tab(sess_tab)])
    w3 = jnp.stack([
        jnp.pad(W[:, 0:d3], ((0, 0), (0, 6))),
        jnp.pad(W[:, d3:2 * d3], ((0, 0), (0, 6))),
        jnp.pad(W[:, 2 * d3:], ((0, 0), (0, 4))),
    ])
    proj = pl.pallas_call(
        _project_tables_kernel,
        out_shape=jax.ShapeDtypeStruct((3, _EMBED_DIM, _EMBED_DIM), jnp.float32),
        in_specs=[
            pl.BlockSpec((3, _EMBED_DIM, 48), lambda: (0, 0, 0)),
            pl.BlockSpec((3, _EMBED_DIM, 48), lambda: (0, 0, 0)),
            pl.BlockSpec((1, _EMBED_DIM), lambda: (0, 0)),
        ],
        out_specs=pl.BlockSpec((3, _EMBED_DIM, _EMBED_DIM), lambda: (0, 0, 0)),
    )(tabs, w3, b.reshape(1, _EMBED_DIM))
    return proj.reshape(3 * _EMBED_DIM, _EMBED_DIM)


def _approx_bucket_kernel(ta_ref, tr_ref, ts_ref, ia_ref, ir_ref, is_ref):
    for p, (t_ref, o_ref) in enumerate(
            ((ta_ref, ia_ref), (tr_ref, ir_ref), (ts_ref, is_ref))):
        f = jnp.maximum(t_ref[:], 1).astype(jnp.float32)
        y = jnp.log(f) * np.float32(_LOG_SCALE)
        b0 = jnp.clip(y.astype(jnp.int32), 0, _NUM_BUCKETS - 1)
        o_ref[:] = b0 + np.int32(p * _EMBED_DIM)


def _approx_buckets(ta, tr, ts, T):
    rows = T // _EMBED_DIM
    br = 800
    grid = (rows // br,)
    spec = pl.BlockSpec((br, _EMBED_DIM), lambda i: (i, 0))
    sh = jax.ShapeDtypeStruct((rows, _EMBED_DIM), jnp.int32)
    r2 = lambda x: x.reshape(rows, _EMBED_DIM)
    ia, ir, is_ = pl.pallas_call(
        _approx_bucket_kernel,
        grid=grid,
        out_shape=(sh, sh, sh),
        in_specs=[spec, spec, spec],
        out_specs=(spec, spec, spec),
        compiler_params=pltpu.CompilerParams(
            dimension_semantics=("arbitrary",)),
    )(r2(ta), r2(tr), r2(ts))
    return ia.reshape(T), ir.reshape(T), is_.reshape(T)


_NC, _NS, _NW = 2, 16, 32
_CHUNK = 128
_BT = 1024
_NCH = _BT // _CHUNK


def _sc_body(p_hbm, ia_hbm, ir_hbm, is_hbm, ta_hbm, tr_hbm, ts_hbm, th_hbm,
             out_hbm, th_v, ib, ga0, gr0, gs0, ga1, gr1, gs1,
             isem, gsem0, gsem1, osem0, osem1):
    T = out_hbm.shape[0]
    tpw = T // _NW
    n_batches = tpw // _BT
    wid = lax.axis_index("s") * _NC + lax.axis_index("c")
    base = wid * tpw
    pltpu.sync_copy(th_hbm, th_v)
    srcs = (ia_hbm, ir_hbm, is_hbm, ta_hbm, tr_hbm, ts_hbm)
    gbufs = ((ga0, gr0, gs0), (ga1, gr1, gs1))
    gsems = (gsem0, gsem1)
    osems = (osem0, osem1)

    def fix_batch():
        def vfix(v, _):
            s = pl.ds(v * 16, 16)
            for q in range(3):
                i = ib[q, s]
                t = ib[q + 3, s]
                lo = plsc.load_gather(th_v, [i])
                hi = plsc.load_gather(th_v, [i + 1])
                ib[q, s] = (i + (t >= hi).astype(jnp.int32)
                            - (t < lo).astype(jnp.int32))
            return 0

        lax.fori_loop(0, _BT // 16, vfix, 0)

    def g_copies(k, p):
        sl = pl.ds(k * _CHUNK, _CHUNK)
        return [
            pltpu.make_async_copy(p_hbm.at[ib.at[q, sl]], gbufs[p][q], gsems[p])
            for q in range(3)
        ]

    def out_copy(b, k, p):
        sl = pl.ds(base + b * _BT + k * _CHUNK, _CHUNK)
        return pltpu.make_async_copy(gbufs[p][0], out_hbm.at[sl], osems[p])

    def combine(p):
        ga, gr, gs = gbufs[p]

        def vadd(t, _):
            for j in range(_EMBED_DIM // 16):
                s2 = pl.ds(j * 16, 16)
                plsc.addupdate(ga.at[t, s2], gr[t, s2] + gs[t, s2])
            return 0

        lax.fori_loop(0, _CHUNK, vadd, 0)

    def batch(b, _):
        sl = pl.ds(base + b * _BT, _BT)
        icps = [pltpu.make_async_copy(srcs[q].at[sl], ib.at[q], isem)
                for q in range(6)]
        for cp in icps:
            cp.start()
        for cp in icps:
            cp.wait()
        fix_batch()
        for k in range(_NCH):
            p = k & 1
            if k == 0:
                for cp in g_copies(0, 0):
                    cp.start()
            if k < _NCH - 1:
                if k >= 1:
                    out_copy(b, k - 1, p ^ 1).wait()
                for cp in g_copies(k + 1, p ^ 1):
                    cp.start()
            for cp in g_copies(k, p):
                cp.wait()
            combine(p)
            out_copy(b, k, p).start()
        out_copy(b, _NCH - 2, 0).wait()
        out_copy(b, _NCH - 1, 1).wait()
        return 0

    lax.fori_loop(0, n_batches, batch, 0)


def kernel(timestamps, session_starts, abs_tab, rel_tab, sess_tab, W, b):
    B, L = timestamps.shape
    T = B * L

    ts = timestamps.astype(jnp.int32)
    t_rel = jnp.concatenate(
        [jnp.zeros((B, 1), jnp.int32), ts[:, 1:] - ts[:, :-1]], axis=1)
    t_sess = ts - session_starts.astype(jnp.int32)[:, None]
    ta, tr, tz = ts.reshape(T), t_rel.reshape(T), t_sess.reshape(T)

    p_flat = _projected_tables(abs_tab, rel_tab, sess_tab, W, b)
    ia0, ir0, is0 = _approx_buckets(ta, tr, tz, T)

    th = _thresholds()
    pad = jnp.full((_EMBED_DIM - _NUM_BUCKETS,), _I32_MAX, jnp.int32)
    blk = jnp.concatenate([th, pad])
    th3 = jnp.concatenate([blk, blk, blk, jnp.full((128,), _I32_MAX, jnp.int32)])

    sck = functools.partial(
        pl.kernel,
        out_type=jax.ShapeDtypeStruct((T, _EMBED_DIM), jnp.float32),
        mesh=plsc.VectorSubcoreMesh(core_axis_name="c", subcore_axis_name="s"),
        compiler_params=pltpu.CompilerParams(needs_layout_passes=False),
        scratch_types=[
            pltpu.VMEM((512,), jnp.int32),
            pltpu.VMEM((6, _BT), jnp.int32),
            pltpu.VMEM((_CHUNK, _EMBED_DIM), jnp.float32),
            pltpu.VMEM((_CHUNK, _EMBED_DIM), jnp.float32),
            pltpu.VMEM((_CHUNK, _EMBED_DIM), jnp.float32),
            pltpu.VMEM((_CHUNK, _EMBED_DIM), jnp.float32),
            pltpu.VMEM((_CHUNK, _EMBED_DIM), jnp.float32),
            pltpu.VMEM((_CHUNK, _EMBED_DIM), jnp.float32),
            pltpu.SemaphoreType.DMA,
            pltpu.SemaphoreType.DMA,
            pltpu.SemaphoreType.DMA,
            pltpu.SemaphoreType.DMA,
            pltpu.SemaphoreType.DMA,
        ],
    )(_sc_body)
    out = sck(p_flat, ia0, ir0, is0, ta, tr, tz, th3)
    return out.reshape(B, L, _EMBED_DIM)


# speedup vs baseline: 9.5580x; 9.5580x over previous
import functools
import math

import jax
import jax.numpy as jnp
import numpy as np
from jax import lax
from jax.experimental import pallas as pl
from jax.experimental.pallas import tpu as pltpu
from jax.experimental.pallas import tpu_sc as plsc

_NUM_BUCKETS = 100
_EMBED_DIM = 128
_MAX_VAL = 1000000.0
_LOG_SCALE = (_NUM_BUCKETS - 1) / math.log(_MAX_VAL)
_I32_MAX = np.int32(2**31 - 1)
_I32_MIN = np.int32(-(2**31))

_BASES = np.round(
    np.exp(np.arange(_NUM_BUCKETS) * (math.log(_MAX_VAL) / (_NUM_BUCKETS - 1)))
).astype(np.int64)
_CANDS = (_BASES[:, None] + np.arange(-16, 16)[None, :]).astype(np.int32)


def _quantize_f32(t_i32):
    clamped = jnp.clip(t_i32.astype(jnp.float32), 1.0, None)
    log_times = jnp.log(clamped) / math.log(_MAX_VAL) * (_NUM_BUCKETS - 1)
    return jnp.clip(log_times.astype(jnp.int32), 0, _NUM_BUCKETS - 1)


def _thresholds():
    cands = lax.optimization_barrier(jnp.asarray(_CANDS))
    q = _quantize_f32(cands)
    ok = q >= jnp.arange(_NUM_BUCKETS, dtype=jnp.int32)[:, None]
    th = jnp.min(jnp.where(ok, cands, _I32_MAX), axis=1).astype(jnp.int32)
    return th.at[0].set(_I32_MIN)


def _project_tables_kernel(tabs_ref, w3_ref, b_ref, out_ref):
    for p in range(3):
        acc = lax.dot_general(
            tabs_ref[p], w3_ref[p], (((1,), (1,)), ((), ())),
            preferred_element_type=jnp.float32)
        if p == 0:
            acc = acc + b_ref[:]
        out_ref[p] = acc


def _projected_tables(abs_tab, rel_tab, sess_tab, W, b):
    d3 = _EMBED_DIM // 3

    def pad_tab(t):
        return jnp.pad(t, ((0, _EMBED_DIM - _NUM_BUCKETS), (0, 48 - t.shape[1])))

    tabs = jnp.stack([pad_tab(abs_tab), pad_tab(rel_tab), pad_tab(sess_tab)])
    w3 = jnp.stack([
        jnp.pad(W[:, 0:d3], ((0, 0), (0, 6))),
        jnp.pad(W[:, d3:2 * d3], ((0, 0), (0, 6))),
        jnp.pad(W[:, 2 * d3:], ((0, 0), (0, 4))),
    ])
    proj = pl.pallas_call(
        _project_tables_kernel,
        out_shape=jax.ShapeDtypeStruct((3, _EMBED_DIM, _EMBED_DIM), jnp.float32),
        in_specs=[
            pl.BlockSpec((3, _EMBED_DIM, 48), lambda: (0, 0, 0)),
            pl.BlockSpec((3, _EMBED_DIM, 48), lambda: (0, 0, 0)),
            pl.BlockSpec((1, _EMBED_DIM), lambda: (0, 0)),
        ],
        out_specs=pl.BlockSpec((3, _EMBED_DIM, _EMBED_DIM), lambda: (0, 0, 0)),
    )(tabs, w3, b.reshape(1, _EMBED_DIM))
    return proj.reshape(3 * _EMBED_DIM, _EMBED_DIM)


def _approx_bucket_kernel(ta_ref, tr_ref, ts_ref, ia_ref, ir_ref, is_ref):
    for p, (t_ref, o_ref) in enumerate(
            ((ta_ref, ia_ref), (tr_ref, ir_ref), (ts_ref, is_ref))):
        f = jnp.maximum(t_ref[:], 1).astype(jnp.float32)
        y = jnp.log(f) * np.float32(_LOG_SCALE)
        b0 = jnp.clip(y.astype(jnp.int32), 0, _NUM_BUCKETS - 1)
        o_ref[:] = b0 + np.int32(p * _EMBED_DIM)


def _approx_buckets(ta, tr, ts, T):
    rows = T // _EMBED_DIM
    br = 800
    grid = (rows // br,)
    spec = pl.BlockSpec((br, _EMBED_DIM), lambda i: (i, 0))
    sh = jax.ShapeDtypeStruct((rows, _EMBED_DIM), jnp.int32)
    r2 = lambda x: x.reshape(rows, _EMBED_DIM)
    ia, ir, is_ = pl.pallas_call(
        _approx_bucket_kernel,
        grid=grid,
        out_shape=(sh, sh, sh),
        in_specs=[spec, spec, spec],
        out_specs=(spec, spec, spec),
        compiler_params=pltpu.CompilerParams(
            dimension_semantics=("arbitrary",)),
    )(r2(ta), r2(tr), r2(ts))
    return ia.reshape(T), ir.reshape(T), is_.reshape(T)


_NC, _NS, _NW = 2, 16, 32
_CHUNK = 128
_BT = 1024
_NCH = _BT // _CHUNK


def _sc_body(p_hbm, ia_hbm, ir_hbm, is_hbm, ta_hbm, tr_hbm, ts_hbm, th_hbm,
             out_hbm, th_v, iav, irv, isv, tav, trv, tsv, p_v, ov0, ov1,
             isem, osem0, osem1):
    T = out_hbm.shape[0]
    tpw = T // _NW
    n_batches = tpw // _BT
    wid = lax.axis_index("s") * _NC + lax.axis_index("c")
    base = wid * tpw
    pltpu.sync_copy(th_hbm, th_v)
    pltpu.sync_copy(p_hbm, p_v)
    srcs = (ia_hbm, ir_hbm, is_hbm, ta_hbm, tr_hbm, ts_hbm)
    ibufs = (iav, irv, isv, tav, trv, tsv)
    obufs = (ov0, ov1)
    osems = (osem0, osem1)
    iota16 = lax.iota(jnp.int32, 16)

    def fix_batch():
        def vfix(v, _):
            s = pl.ds(v * 16, 16)
            for q in range(3):
                i = ibufs[q][s]
                t = ibufs[q + 3][s]
                lo = plsc.load_gather(th_v, [i])
                hi = plsc.load_gather(th_v, [i + 1])
                ibufs[q][s] = (i + (t >= hi).astype(jnp.int32)
                               - (t < lo).astype(jnp.int32))
            return 0

        lax.fori_loop(0, _BT // 16, vfix, 0)

    def gather_chunk(k, p):
        ov = obufs[p]

        def token(t, _):
            pos = jnp.full((16,), k * _CHUNK + t, jnp.int32)
            addrs = []
            for q in range(3):
                r = plsc.load_gather(ibufs[q], [pos])
                addrs.append((r << 7) + iota16)
            aa, ar, az = addrs
            for j in range(_EMBED_DIM // 16):
                va = plsc.load_gather(p_v, [aa])
                vr = plsc.load_gather(p_v, [ar])
                vs = plsc.load_gather(p_v, [az])
                ov[t, pl.ds(j * 16, 16)] = va + vr + vs
                if j < _EMBED_DIM // 16 - 1:
                    aa = aa + 16
                    ar = ar + 16
                    az = az + 16
            return 0

        lax.fori_loop(0, _CHUNK, token, 0)

    def out_copy(b, k, p):
        sl = pl.ds(base + b * _BT + k * _CHUNK, _CHUNK)
        return pltpu.make_async_copy(obufs[p], out_hbm.at[sl], osems[p])

    def batch(b, _):
        sl = pl.ds(base + b * _BT, _BT)
        icps = [pltpu.make_async_copy(srcs[q].at[sl], ibufs[q], isem)
                for q in range(6)]
        for cp in icps:
            cp.start()
        for cp in icps:
            cp.wait()
        fix_batch()
        for k in range(_NCH):
            p = k & 1
            if k >= 2:
                out_copy(b, k - 2, p).wait()
            gather_chunk(k, p)
            out_copy(b, k, p).start()
        out_copy(b, _NCH - 2, 0).wait()
        out_copy(b, _NCH - 1, 1).wait()
        return 0

    lax.fori_loop(0, n_batches, batch, 0)


def kernel(timestamps, session_starts, abs_tab, rel_tab, sess_tab, W, b):
    B, L = timestamps.shape
    T = B * L

    ts = timestamps.astype(jnp.int32)
    t_rel = jnp.concatenate(
        [jnp.zeros((B, 1), jnp.int32), ts[:, 1:] - ts[:, :-1]], axis=1)
    t_sess = ts - session_starts.astype(jnp.int32)[:, None]
    ta, tr, tz = ts.reshape(T), t_rel.reshape(T), t_sess.reshape(T)

    p_flat = _projected_tables(abs_tab, rel_tab, sess_tab, W, b)
    ia0, ir0, is0 = _approx_buckets(ta, tr, tz, T)

    th = _thresholds()
    pad = jnp.full((_EMBED_DIM - _NUM_BUCKETS,), _I32_MAX, jnp.int32)
    blk = jnp.concatenate([th, pad])
    th3 = jnp.concatenate([blk, blk, blk, jnp.full((128,), _I32_MAX, jnp.int32)])

    sck = functools.partial(
        pl.kernel,
        out_type=jax.ShapeDtypeStruct((T, _EMBED_DIM), jnp.float32),
        mesh=plsc.VectorSubcoreMesh(core_axis_name="c", subcore_axis_name="s"),
        compiler_params=pltpu.CompilerParams(needs_layout_passes=False),
        scratch_types=[
            pltpu.VMEM((512,), jnp.int32),
            pltpu.VMEM((_BT,), jnp.int32),
            pltpu.VMEM((_BT,), jnp.int32),
            pltpu.VMEM((_BT,), jnp.int32),
            pltpu.VMEM((_BT,), jnp.int32),
            pltpu.VMEM((_BT,), jnp.int32),
            pltpu.VMEM((_BT,), jnp.int32),
            pltpu.VMEM((3 * _EMBED_DIM * _EMBED_DIM,), jnp.float32),
            pltpu.VMEM((_CHUNK, _EMBED_DIM), jnp.float32),
            pltpu.VMEM((_CHUNK, _EMBED_DIM), jnp.float32),
            pltpu.SemaphoreType.DMA,
            pltpu.SemaphoreType.DMA,
            pltpu.SemaphoreType.DMA,
        ],
    )(_sc_body)
    out = sck(p_flat.reshape(3 * _EMBED_DIM * _EMBED_DIM), ia0, ir0, is0,
              ta, tr, tz, th3)
    return out.reshape(B, L, _EMBED_DIM)


# speedup vs baseline: 11.0845x; 1.1597x over previous
import functools
import math

import jax
import jax.numpy as jnp
import numpy as np
from jax import lax
from jax.experimental import pallas as pl
from jax.experimental.pallas import tpu as pltpu
from jax.experimental.pallas import tpu_sc as plsc

_NUM_BUCKETS = 100
_EMBED_DIM = 128
_MAX_VAL = 1000000.0
_LOG_SCALE = (_NUM_BUCKETS - 1) / math.log(_MAX_VAL)
_I32_MAX = np.int32(2**31 - 1)
_I32_MIN = np.int32(-(2**31))

_BASES = np.round(
    np.exp(np.arange(_NUM_BUCKETS) * (math.log(_MAX_VAL) / (_NUM_BUCKETS - 1)))
).astype(np.int64)
_CANDS = (_BASES[:, None] + np.arange(-16, 16)[None, :]).astype(np.int32)


def _quantize_f32(t_i32):
    clamped = jnp.clip(t_i32.astype(jnp.float32), 1.0, None)
    log_times = jnp.log(clamped) / math.log(_MAX_VAL) * (_NUM_BUCKETS - 1)
    return jnp.clip(log_times.astype(jnp.int32), 0, _NUM_BUCKETS - 1)


def _thresholds():
    cands = lax.optimization_barrier(jnp.asarray(_CANDS))
    q = _quantize_f32(cands)
    ok = q >= jnp.arange(_NUM_BUCKETS, dtype=jnp.int32)[:, None]
    th = jnp.min(jnp.where(ok, cands, _I32_MAX), axis=1).astype(jnp.int32)
    return th.at[0].set(_I32_MIN)


def _project_tables_kernel(tabs_ref, w3_ref, b_ref, out_ref):
    for p in range(3):
        acc = lax.dot_general(
            tabs_ref[p], w3_ref[p], (((1,), (1,)), ((), ())),
            preferred_element_type=jnp.float32)
        if p == 0:
            acc = acc + b_ref[:]
        out_ref[p] = acc


def _projected_tables(abs_tab, rel_tab, sess_tab, W, b):
    d3 = _EMBED_DIM // 3

    def pad_tab(t):
        return jnp.pad(t, ((0, _EMBED_DIM - _NUM_BUCKETS), (0, 48 - t.shape[1])))

    tabs = jnp.stack([pad_tab(abs_tab), pad_tab(rel_tab), pad_tab(sess_tab)])
    w3 = jnp.stack([
        jnp.pad(W[:, 0:d3], ((0, 0), (0, 6))),
        jnp.pad(W[:, d3:2 * d3], ((0, 0), (0, 6))),
        jnp.pad(W[:, 2 * d3:], ((0, 0), (0, 4))),
    ])
    proj = pl.pallas_call(
        _project_tables_kernel,
        out_shape=jax.ShapeDtypeStruct((3, _EMBED_DIM, _EMBED_DIM), jnp.float32),
        in_specs=[
            pl.BlockSpec((3, _EMBED_DIM, 48), lambda: (0, 0, 0)),
            pl.BlockSpec((3, _EMBED_DIM, 48), lambda: (0, 0, 0)),
            pl.BlockSpec((1, _EMBED_DIM), lambda: (0, 0)),
        ],
        out_specs=pl.BlockSpec((3, _EMBED_DIM, _EMBED_DIM), lambda: (0, 0, 0)),
    )(tabs, w3, b.reshape(1, _EMBED_DIM))
    return proj.reshape(3 * _EMBED_DIM, _EMBED_DIM)


def _approx_bucket_kernel(ta_ref, tr_ref, ts_ref, ia_ref, ir_ref, is_ref):
    for p, (t_ref, o_ref) in enumerate(
            ((ta_ref, ia_ref), (tr_ref, ir_ref), (ts_ref, is_ref))):
        f = jnp.maximum(t_ref[:], 1).astype(jnp.float32)
        y = jnp.log(f) * np.float32(_LOG_SCALE)
        b0 = jnp.clip(y.astype(jnp.int32), 0, _NUM_BUCKETS - 1)
        o_ref[:] = b0 + np.int32(p * _EMBED_DIM)


def _approx_buckets(ta, tr, ts, T):
    rows = T // _EMBED_DIM
    br = 800
    grid = (rows // br,)
    spec = pl.BlockSpec((br, _EMBED_DIM), lambda i: (i, 0))
    sh = jax.ShapeDtypeStruct((rows, _EMBED_DIM), jnp.int32)
    r2 = lambda x: x.reshape(rows, _EMBED_DIM)
    ia, ir, is_ = pl.pallas_call(
        _approx_bucket_kernel,
        grid=grid,
        out_shape=(sh, sh, sh),
        in_specs=[spec, spec, spec],
        out_specs=(spec, spec, spec),
        compiler_params=pltpu.CompilerParams(
            dimension_semantics=("arbitrary",)),
    )(r2(ta), r2(tr), r2(ts))
    return ia.reshape(T), ir.reshape(T), is_.reshape(T)


_NC, _NS, _NW = 2, 16, 32
_CHUNK = 128
_BT = 1024
_NCH = _BT // _CHUNK


def _sc_body(p_hbm, ia_hbm, ir_hbm, is_hbm, ta_hbm, tr_hbm, ts_hbm, th_hbm,
             out_hbm, th_v, iav, irv, isv, tav, trv, tsv, p_v, ov0, ov1,
             isem, osem0, osem1):
    T = out_hbm.shape[0]
    tpw = T // _NW
    n_batches = tpw // _BT
    wid = lax.axis_index("s") * _NC + lax.axis_index("c")
    base = wid * tpw
    pltpu.sync_copy(th_hbm, th_v)
    pltpu.sync_copy(p_hbm, p_v)
    srcs = (ia_hbm, ir_hbm, is_hbm, ta_hbm, tr_hbm, ts_hbm)
    ibufs = (iav, irv, isv, tav, trv, tsv)
    obufs = (ov0, ov1)
    osems = (osem0, osem1)
    iota16 = lax.iota(jnp.int32, 16)

    def vsplat(vec, sel):
        return lax.gather(
            vec, sel[:, None],
            lax.GatherDimensionNumbers(
                offset_dims=(), collapsed_slice_dims=(0,),
                start_index_map=(0,)),
            (1,), mode=lax.GatherScatterMode.PROMISE_IN_BOUNDS)

    def fix_batch():
        def vfix(v, _):
            s = pl.ds(v * 16, 16)
            for q in range(3):
                i = ibufs[q][s]
                t = ibufs[q + 3][s]
                lo = plsc.load_gather(th_v, [i])
                hi = plsc.load_gather(th_v, [i + 1])
                ibufs[q][s] = (i + (t >= hi).astype(jnp.int32)
                               - (t < lo).astype(jnp.int32))
            return 0

        lax.fori_loop(0, _BT // 16, vfix, 0)

    def gather_chunk(k, p):
        ov = obufs[p]

        def group(g, _):
            s16 = pl.ds(k * _CHUNK + g * 16, 16)
            rv0 = ibufs[0][s16] << 7
            rv1 = ibufs[1][s16] << 7
            rv2 = ibufs[2][s16] << 7

            def token(tt, _):
                sel = jnp.full((16,), tt, jnp.int32)
                aa = vsplat(rv0, sel) + iota16
                ar = vsplat(rv1, sel) + iota16
                az = vsplat(rv2, sel) + iota16
                t = g * 16 + tt
                for j in range(_EMBED_DIM // 16):
                    va = plsc.load_gather(p_v, [aa])
                    vr = plsc.load_gather(p_v, [ar])
                    vs = plsc.load_gather(p_v, [az])
                    ov[t, pl.ds(j * 16, 16)] = va + vr + vs
                    if j < _EMBED_DIM // 16 - 1:
                        aa = aa + 16
                        ar = ar + 16
                        az = az + 16
                return 0

            lax.fori_loop(0, 16, token, 0, unroll=4)
            return 0

        lax.fori_loop(0, _CHUNK // 16, group, 0)

    def out_copy(b, k, p):
        sl = pl.ds(base + b * _BT + k * _CHUNK, _CHUNK)
        return pltpu.make_async_copy(obufs[p], out_hbm.at[sl], osems[p])

    def batch(b, _):
        sl = pl.ds(base + b * _BT, _BT)
        icps = [pltpu.make_async_copy(srcs[q].at[sl], ibufs[q], isem)
                for q in range(6)]
        for cp in icps:
            cp.start()
        for cp in icps:
            cp.wait()
        fix_batch()
        for k in range(_NCH):
            p = k & 1
            if k >= 2:
                out_copy(b, k - 2, p).wait()
            gather_chunk(k, p)
            out_copy(b, k, p).start()
        out_copy(b, _NCH - 2, 0).wait()
        out_copy(b, _NCH - 1, 1).wait()
        return 0

    lax.fori_loop(0, n_batches, batch, 0)


def kernel(timestamps, session_starts, abs_tab, rel_tab, sess_tab, W, b):
    B, L = timestamps.shape
    T = B * L

    ts = timestamps.astype(jnp.int32)
    t_rel = jnp.concatenate(
        [jnp.zeros((B, 1), jnp.int32), ts[:, 1:] - ts[:, :-1]], axis=1)
    t_sess = ts - session_starts.astype(jnp.int32)[:, None]
    ta, tr, tz = ts.reshape(T), t_rel.reshape(T), t_sess.reshape(T)

    p_flat = _projected_tables(abs_tab, rel_tab, sess_tab, W, b)
    ia0, ir0, is0 = _approx_buckets(ta, tr, tz, T)

    th = _thresholds()
    pad = jnp.full((_EMBED_DIM - _NUM_BUCKETS,), _I32_MAX, jnp.int32)
    blk = jnp.concatenate([th, pad])
    th3 = jnp.concatenate([blk, blk, blk, jnp.full((128,), _I32_MAX, jnp.int32)])

    sck = functools.partial(
        pl.kernel,
        out_type=jax.ShapeDtypeStruct((T, _EMBED_DIM), jnp.float32),
        mesh=plsc.VectorSubcoreMesh(core_axis_name="c", subcore_axis_name="s"),
        compiler_params=pltpu.CompilerParams(needs_layout_passes=False),
        scratch_types=[
            pltpu.VMEM((512,), jnp.int32),
            pltpu.VMEM((_BT,), jnp.int32),
            pltpu.VMEM((_BT,), jnp.int32),
            pltpu.VMEM((_BT,), jnp.int32),
            pltpu.VMEM((_BT,), jnp.int32),
            pltpu.VMEM((_BT,), jnp.int32),
            pltpu.VMEM((_BT,), jnp.int32),
            pltpu.VMEM((3 * _EMBED_DIM * _EMBED_DIM,), jnp.float32),
            pltpu.VMEM((_CHUNK, _EMBED_DIM), jnp.float32),
            pltpu.VMEM((_CHUNK, _EMBED_DIM), jnp.float32),
            pltpu.SemaphoreType.DMA,
            pltpu.SemaphoreType.DMA,
            pltpu.SemaphoreType.DMA,
        ],
    )(_sc_body)
    out = sck(p_flat.reshape(3 * _EMBED_DIM * _EMBED_DIM), ia0, ir0, is0,
              ta, tr, tz, th3)
    return out.reshape(B, L, _EMBED_DIM)


# speedup vs baseline: 46.7835x; 4.2206x over previous
import functools
import math

import jax
import jax.numpy as jnp
import numpy as np
from jax import lax
from jax.experimental import pallas as pl
from jax.experimental.pallas import tpu as pltpu
from jax.experimental.pallas import tpu_sc as plsc

_NUM_BUCKETS = 100
_EMBED_DIM = 128
_MAX_VAL = 1000000.0
_LOG_SCALE = (_NUM_BUCKETS - 1) / math.log(_MAX_VAL)
_I32_MAX = np.int32(2**31 - 1)
_I32_MIN = np.int32(-(2**31))

_BASES = np.round(
    np.exp(np.arange(_NUM_BUCKETS) * (math.log(_MAX_VAL) / (_NUM_BUCKETS - 1)))
).astype(np.int64)
_CANDS = (_BASES[:, None] + np.arange(-16, 16)[None, :]).astype(np.int32)


def _quantize_f32(t_i32):
    clamped = jnp.clip(t_i32.astype(jnp.float32), 1.0, None)
    log_times = jnp.log(clamped) / math.log(_MAX_VAL) * (_NUM_BUCKETS - 1)
    return jnp.clip(log_times.astype(jnp.int32), 0, _NUM_BUCKETS - 1)


def _thresholds():
    cands = lax.optimization_barrier(jnp.asarray(_CANDS))
    q = _quantize_f32(cands)
    ok = q >= jnp.arange(_NUM_BUCKETS, dtype=jnp.int32)[:, None]
    th = jnp.min(jnp.where(ok, cands, _I32_MAX), axis=1).astype(jnp.int32)
    return th.at[0].set(_I32_MIN)


def _project_tables_kernel(tabs_ref, w3_ref, b_ref, out_ref):
    for p in range(3):
        acc = lax.dot_general(
            tabs_ref[p], w3_ref[p], (((1,), (1,)), ((), ())),
            preferred_element_type=jnp.float32)
        if p == 0:
            acc = acc + b_ref[:]
        out_ref[p] = acc


def _approx_bucket_kernel(ts_ref, ss_ref, tabs_ref, w3_ref, b_ref,
                          ia_ref, ir_ref, is_ref, tr_ref, tz_ref,
                          proj_ref):
    @pl.when(pl.program_id(0) == 0)
    def _():
        _project_tables_kernel(tabs_ref, w3_ref, b_ref, proj_ref)

    ts = ts_ref[:]
    tr = jnp.concatenate(
        [jnp.zeros_like(ts[:, :1]), ts[:, 1:] - ts[:, :-1]], axis=1)
    tz = ts - ss_ref[:]
    for p, (t, o_ref, t_ref) in enumerate(
            ((ts, ia_ref, None), (tr, ir_ref, tr_ref), (tz, is_ref, tz_ref))):
        f = jnp.maximum(t, 1).astype(jnp.float32)
        y = jnp.log(f) * np.float32(_LOG_SCALE)
        b0 = jnp.clip(y.astype(jnp.int32), 0, _NUM_BUCKETS - 1)
        o_ref[:] = b0 + np.int32(p * _EMBED_DIM)
        if t_ref is not None:
            t_ref[:] = t


def _approx_buckets(ts, session_starts, tabs, w3, b, B, L):
    br = 512
    grid = (B // br,)
    spec = pl.BlockSpec((br, L), lambda i: (i, 0))
    sspec = pl.BlockSpec((br, 1), lambda i: (i, 0))
    sh = jax.ShapeDtypeStruct((B, L), jnp.int32)
    outs = pl.pallas_call(
        _approx_bucket_kernel,
        grid=grid,
        out_shape=(sh,) * 5 + (
            jax.ShapeDtypeStruct((3, _EMBED_DIM, _EMBED_DIM), jnp.float32),),
        in_specs=[
            spec, sspec,
            pl.BlockSpec((3, _EMBED_DIM, 48), lambda i: (0, 0, 0)),
            pl.BlockSpec((3, _EMBED_DIM, 48), lambda i: (0, 0, 0)),
            pl.BlockSpec((1, _EMBED_DIM), lambda i: (0, 0)),
        ],
        out_specs=(spec,) * 5 + (
            pl.BlockSpec((3, _EMBED_DIM, _EMBED_DIM), lambda i: (0, 0, 0)),),
        compiler_params=pltpu.CompilerParams(
            dimension_semantics=("arbitrary",)),
    )(ts, session_starts.astype(jnp.int32)[:, None], tabs, w3, b)
    return tuple(o.reshape(B * L) for o in outs[:5]) + (outs[5],)


_NC, _NS, _NW = 2, 16, 32
_CHUNK = 128
_BT = 1024
_NCH = _BT // _CHUNK


def _sc_body(p_hbm, ia_hbm, ir_hbm, is_hbm, ta_hbm, tr_hbm, ts_hbm, th_hbm,
             out_hbm, th_v,
             ia0, ir0, is0, ta0, tr0, ts0,
             ia1, ir1, is1, ta1, tr1, ts1,
             p_v, ov0, ov1, isem0, isem1, osem0, osem1):
    T = out_hbm.shape[0]
    tpw = T // _NW
    wid = lax.axis_index("s") * _NC + lax.axis_index("c")
    base = wid * tpw
    pltpu.sync_copy(th_hbm, th_v)
    pltpu.sync_copy(p_hbm, p_v)
    srcs = (ia_hbm, ir_hbm, is_hbm, ta_hbm, tr_hbm, ts_hbm)
    isets = ((ia0, ir0, is0, ta0, tr0, ts0),
             (ia1, ir1, is1, ta1, tr1, ts1))
    isems = (isem0, isem1)
    obufs = (ov0, ov1)
    osems = (osem0, osem1)
    iota16 = lax.iota(jnp.int32, 16)

    def vsplat(vec, sel):
        return lax.gather(
            vec, sel[:, None],
            lax.GatherDimensionNumbers(
                offset_dims=(), collapsed_slice_dims=(0,),
                start_index_map=(0,)),
            (1,), mode=lax.GatherScatterMode.PROMISE_IN_BOUNDS)

    def fix_batch(ibufs):
        def vfix(v, _):
            s = pl.ds(v * 16, 16)
            for q in range(3):
                i = ibufs[q][s]
                t = ibufs[q + 3][s]
                lo = plsc.load_gather(th_v, [i])
                hi = plsc.load_gather(th_v, [i + 1])
                ibufs[q][s] = (i + (t >= hi).astype(jnp.int32)
                               - (t < lo).astype(jnp.int32))
            return 0

        lax.fori_loop(0, _BT // 16, vfix, 0)

    def gather_chunk(ibufs, k, p):
        ov = obufs[p]

        def group(g, _):
            s16 = pl.ds(k * _CHUNK + g * 16, 16)
            rv0 = ibufs[0][s16] << 6
            rv1 = ibufs[1][s16] << 6
            rv2 = ibufs[2][s16] << 6

            @plsc.parallel_loop(0, 16, unroll=4)
            def token(tt):
                sel = jnp.full((16,), tt, jnp.int32)
                aa = vsplat(rv0, sel) + iota16
                ar = vsplat(rv1, sel) + iota16
                az = vsplat(rv2, sel) + iota16
                t = g * 16 + tt
                n = p_v.shape[0]
                for j in range(_EMBED_DIM // 32):
                    pj = p_v.at[pl.ds(j * 16, n - j * 16)]
                    ba = plsc.bitcast(plsc.load_gather(pj, [aa]), jnp.bfloat16)
                    br = plsc.bitcast(plsc.load_gather(pj, [ar]), jnp.bfloat16)
                    bs = plsc.bitcast(plsc.load_gather(pj, [az]), jnp.bfloat16)
                    even, odd = plsc.unpack(
                        ba + br + bs, format=plsc.PackFormat.INTERLEAVED)
                    ov[t, pl.ds(j * 32, 16)] = even
                    ov[t, pl.ds(j * 32 + 16, 16)] = odd

            return 0

        lax.fori_loop(0, _CHUNK // 16, group, 0)

    def out_copy(b, k, p):
        sl = pl.ds(base + b * _BT + k * _CHUNK, _CHUNK)
        return pltpu.make_async_copy(obufs[p], out_hbm.at[sl], osems[p])

    def load_copies(b, s):
        sl = pl.ds(base + b * _BT, _BT)
        return [pltpu.make_async_copy(srcs[q].at[sl], isets[s][q], isems[s])
                for q in range(6)]

    def process(b, s):
        fix_batch(isets[s])
        for k in range(_NCH):
            p = k & 1
            if k >= 2:
                out_copy(b, k - 2, p).wait()
            gather_chunk(isets[s], k, p)
            out_copy(b, k, p).start()
        out_copy(b, _NCH - 2, (_NCH - 2) & 1).wait()
        out_copy(b, _NCH - 1, (_NCH - 1) & 1).wait()

    for cp in load_copies(0, 0):
        cp.start()

    def pair(m, _):
        b0 = 2 * m
        for cp in load_copies(b0 + 1, 1):
            cp.start()
        for cp in load_copies(b0, 0):
            cp.wait()
        process(b0, 0)
        for cp in load_copies(b0 + 2, 0):
            cp.start()
        for cp in load_copies(b0 + 1, 1):
            cp.wait()
        process(b0 + 1, 1)
        return 0

    n_pairs = (tpw // _BT) // 2
    lax.fori_loop(0, n_pairs, pair, 0)
    last = tpw // _BT - 1
    for cp in load_copies(last, 0):
        cp.wait()
    process(last, 0)


def kernel(timestamps, session_starts, abs_tab, rel_tab, sess_tab, W, b):
    B, L = timestamps.shape
    T = B * L

    ts = timestamps.astype(jnp.int32)
    d3 = _EMBED_DIM // 3

    def pad_tab(t):
        return jnp.pad(t, ((0, _EMBED_DIM - _NUM_BUCKETS), (0, 48 - t.shape[1])))

    tabs = jnp.stack([pad_tab(abs_tab), pad_tab(rel_tab), pad_tab(sess_tab)])
    w3 = jnp.stack([
        jnp.pad(W[:, 0:d3], ((0, 0), (0, 6))),
        jnp.pad(W[:, d3:2 * d3], ((0, 0), (0, 6))),
        jnp.pad(W[:, 2 * d3:], ((0, 0), (0, 4))),
    ])
    ia0, ir0, is0, tr, tz, proj = _approx_buckets(
        ts, session_starts, tabs, w3, b.reshape(1, _EMBED_DIM), B, L)
    ta = ts.reshape(T)
    p_flat = proj.reshape(3 * _EMBED_DIM, _EMBED_DIM)

    th = _thresholds()
    pad = jnp.full((_EMBED_DIM - _NUM_BUCKETS,), _I32_MAX, jnp.int32)
    blk = jnp.concatenate([th, pad])
    th3 = jnp.concatenate([blk, blk, blk, jnp.full((128,), _I32_MAX, jnp.int32)])

    sck = functools.partial(
        pl.kernel,
        out_type=jax.ShapeDtypeStruct((T, _EMBED_DIM), jnp.float32),
        mesh=plsc.VectorSubcoreMesh(core_axis_name="c", subcore_axis_name="s"),
        compiler_params=pltpu.CompilerParams(needs_layout_passes=False),
        scratch_types=(
            [pltpu.VMEM((512,), jnp.int32)]
            + [pltpu.VMEM((_BT,), jnp.int32) for _ in range(12)]
            + [
                pltpu.VMEM((3 * _EMBED_DIM * _EMBED_DIM // 2,), jnp.int32),
                pltpu.VMEM((_CHUNK, _EMBED_DIM), jnp.float32),
                pltpu.VMEM((_CHUNK, _EMBED_DIM), jnp.float32),
                pltpu.SemaphoreType.DMA,
                pltpu.SemaphoreType.DMA,
                pltpu.SemaphoreType.DMA,
                pltpu.SemaphoreType.DMA,
            ]
        ),
    )(_sc_body)
    p_perm = p_flat.reshape(3 * _EMBED_DIM, 4, 2, 16).transpose(0, 1, 3, 2)
    p_packed = jax.lax.bitcast_convert_type(
        p_perm.astype(jnp.bfloat16), jnp.int32)
    out = sck(p_packed.reshape(3 * _EMBED_DIM * _EMBED_DIM // 2), ia0, ir0,
              is0, ta, tr, tz, th3)
    return out.reshape(B, L, _EMBED_DIM)
